# Initial kernel scaffold; baseline (speedup 1.0000x reference)
import jax
import jax.numpy as jnp
from jax.experimental import pallas as pl


def kernel(x, edge_index, edge_attr, params):
    raise NotImplementedError("write your pallas kernel here")


# speedup vs baseline: 2.0879x; 2.0879x over previous
import functools

import jax
import jax.numpy as jnp
from jax import lax
from jax.experimental import pallas as pl
from jax.experimental.pallas import tpu as pltpu
from jax.experimental.pallas import tpu_sc as plsc

_DIM = 16
_N = 10000
_E = 160000
_NW = 32
_CH = 128
_CPW = 40
_EPW = _CH * _CPW
_EP = _NW * _EPW
_NACC = 10016
_RPS = _NACC // 16
_T = 2048
_G = _EP // _T

_f32 = jnp.float32


def _mesh():
    return plsc.VectorSubcoreMesh(core_axis_name="c", subcore_axis_name="s",
                                  num_cores=2, num_subcores=16)


def _sc_gather_pair(table, rowi, coli):
    @functools.partial(
        pl.kernel,
        out_type=(jax.ShapeDtypeStruct((_EP, _DIM), _f32),
                  jax.ShapeDtypeStruct((_EP, _DIM), _f32)),
        mesh=_mesh(),
        scratch_types=(pltpu.VMEM((_CPW, _CH), jnp.int32),
                       pltpu.VMEM((_EPW, _DIM), _f32),
                       pltpu.SemaphoreType.DMA),
        compiler_params=pltpu.CompilerParams(use_tc_tiling_on_sc=False),
    )
    def gk(table_h, rowi_h, coli_h, orow_h, ocol_h, idx_v, buf_v, sem):
        cid = lax.axis_index("c")
        sid = lax.axis_index("s")
        w = sid * 2 + cid
        rbase = w * _CPW
        ebase = w * _EPW

        def one(src_h, dst_h):
            pltpu.sync_copy(src_h.at[pl.ds(rbase, _CPW)], idx_v)

            def chunk8(i, carry):
                b = i * 8
                hs = []
                for k in range(8):
                    hs.append(pltpu.async_copy(
                        table_h.at[idx_v.at[b + k]],
                        buf_v.at[pl.ds((b + k) * _CH, _CH)], sem))
                for h in hs:
                    h.wait()
                return carry

            lax.fori_loop(0, _CPW // 8, chunk8, 0)
            pltpu.sync_copy(buf_v, dst_h.at[pl.ds(ebase, _EPW)])

        one(rowi_h, orow_h)
        one(coli_h, ocol_h)

    return gk(table, rowi, coli)


def _sc_scatter_add(msg, coli):
    @functools.partial(
        pl.kernel,
        out_type=jax.ShapeDtypeStruct((2, _NACC, _DIM), _f32),
        mesh=_mesh(),
        scratch_types=(pltpu.VMEM((_CPW, _CH), jnp.int32),
                       pltpu.VMEM((_EPW, _DIM), _f32),
                       pltpu.VMEM((_RPS, _DIM), _f32),
                       pltpu.VMEM_SHARED((_NACC, _DIM), _f32),
                       pltpu.SemaphoreType.DMA),
        compiler_params=pltpu.CompilerParams(use_tc_tiling_on_sc=False),
    )
    def sk(msg_h, coli_h, out_h, idx_v, buf_v, zbuf_v, acc_sh, sem):
        cid = lax.axis_index("c")
        sid = lax.axis_index("s")
        w = sid * 2 + cid

        def zrow(i, carry):
            zbuf_v[i, :] = jnp.zeros((_DIM,), _f32)
            return carry

        lax.fori_loop(0, _RPS, zrow, 0)
        pltpu.sync_copy(zbuf_v, acc_sh.at[pl.ds(sid * _RPS, _RPS)])
        plsc.subcore_barrier()

        pltpu.sync_copy(coli_h.at[pl.ds(w * _CPW, _CPW)], idx_v)
        pltpu.sync_copy(msg_h.at[pl.ds(w * _EPW, _EPW)], buf_v)

        def chunk(i, carry):
            pltpu.sync_copy(buf_v.at[pl.ds(i * _CH, _CH)],
                            acc_sh.at[idx_v.at[i]], add=True)
            return carry

        lax.fori_loop(0, _CPW, chunk, 0)
        plsc.subcore_barrier()

        @pl.when(sid == 0)
        def _():
            pltpu.sync_copy(acc_sh, out_h.at[cid])

    return sk(msg, coli)


def _layernorm(h, g, b):
    m = jnp.mean(h, axis=-1, keepdims=True)
    v = jnp.mean((h - m) ** 2, axis=-1, keepdims=True)
    return (h - m) * lax.rsqrt(v + 1e-5) * g + b


def _wmlp(e, cw1, cb1, cw2, cb2, cw3, cb3):
    h = jnp.maximum(jnp.dot(e, cw1, preferred_element_type=_f32) + cb1, 0.0)
    h = jnp.maximum(jnp.dot(h, cw2, preferred_element_type=_f32) + cb2, 0.0)
    return jnp.maximum(jnp.dot(h, cw3, preferred_element_type=_f32) + cb3, 0.0)


def _msg_contract(xn, w3o):
    acc = xn[:, 0:1] * w3o[:, 0:_DIM]
    for i in range(1, _DIM):
        acc = acc + xn[:, i:i + 1] * w3o[:, _DIM * i:_DIM * (i + 1)]
    return acc


def _small_edge(xr, xc, ea, sw1r, sw1c, sw1e, sb1, g, be, sw2, sb2):
    h = (jnp.dot(xr, sw1r, preferred_element_type=_f32)
         + jnp.dot(xc, sw1c, preferred_element_type=_f32)
         + jnp.dot(ea, sw1e, preferred_element_type=_f32) + sb1)
    h = _layernorm(jnp.maximum(h, 0.0), g, be)
    return jnp.dot(h, sw2, preferred_element_type=_f32) + sb2


def _full_spec(a):
    return pl.BlockSpec(a.shape, lambda i: (0,) * a.ndim)


def _tile_spec(d):
    return pl.BlockSpec((_T, d), lambda i: (i, 0))


def _ek_call(body, tiled_ins, full_ins, out_dims):
    in_specs = ([_tile_spec(a.shape[1]) for a in tiled_ins]
                + [_full_spec(a) for a in full_ins])
    out_shape = [jax.ShapeDtypeStruct((_EP, d), _f32) for d in out_dims]
    out_specs = [_tile_spec(d) for d in out_dims]
    res = pl.pallas_call(
        body, grid=(_G,), in_specs=in_specs,
        out_specs=out_specs, out_shape=out_shape,
    )(*tiled_ins, *full_ins)
    return res


def _ek0_body(ea_ref, xr_ref, st_ref, cw1, cb1, cw2, cb2, cw3, cb3, msg_ref):
    e = jnp.abs(ea_ref[...])
    w3o = _wmlp(e, cw1[...], cb1[...], cw2[...], cb2[...], cw3[...], cb3[...])
    xn = (xr_ref[:, 0:1] - st_ref[0, 0]) * st_ref[1, 0]
    msg_ref[...] = w3o * xn


def _ek_mid_body(ea_ref, xr_ref, xc_ref, st_ref,
                 sw1r, sw1c, sw1e, sb1, g, be, sw2, sb2,
                 cw1, cb1, cw2, cb2, cw3, cb3,
                 ea_out_ref, msg_ref, *, out_ch):
    xr = xr_ref[...]
    ea_prev = ea_ref[...]
    ea_new = jnp.maximum(
        _small_edge(xr, xc_ref[...], ea_prev, sw1r[...], sw1c[...], sw1e[...],
                    sb1[...], g[...], be[...], sw2[...], sb2[...]), 0.0) + ea_prev
    ea_out_ref[...] = ea_new
    w3o = _wmlp(ea_new, cw1[...], cb1[...], cw2[...], cb2[...], cw3[...], cb3[...])
    xn = (xr - st_ref[0:1, :]) * st_ref[1:2, :]
    if out_ch == _DIM:
        msg_ref[...] = _msg_contract(xn, w3o)
    else:
        m = jnp.sum(xn * w3o, axis=-1, keepdims=True)
        msg_ref[...] = jnp.concatenate(
            [m, jnp.zeros((m.shape[0], _DIM - 1), _f32)], axis=1)


def _ek_fin_body(ea_ref, xr_ref, xc_ref,
                 sw1r, sw1c, sw1e, sb1, g, be, sw2, sb2, ea_out_ref):
    h = _small_edge(xr_ref[:, 0:1], xc_ref[:, 0:1], ea_ref[...],
                    sw1r[...], sw1c[...], sw1e[...], sb1[...], g[...],
                    be[...], sw2[...], sb2[...])
    ea_out_ref[...] = jnp.maximum(h, 0.0)


def _stats(x):
    m = jnp.mean(x, axis=0, keepdims=True)
    v = jnp.mean((x - m) ** 2, axis=0, keepdims=True)
    return jnp.concatenate([m, lax.rsqrt(v + 1e-5)], axis=0)


def _s0_body(x_ref, st_ref):
    st = _stats(x_ref[...])
    st_ref[...] = jnp.broadcast_to(st, (2, _DIM))


def _nk0_body(p_ref, x_ref, st_ref, root, bias, xn_ref, stn_ref):
    agg = (p_ref[0] + p_ref[1])[:_N]
    x0 = x_ref[...]
    xn0 = (x0 - st_ref[0, 0]) * st_ref[1, 0]
    x1 = jnp.maximum(agg + xn0 * root[...] + bias[...], 0.0) + x0
    xn_ref[...] = x1
    stn_ref[...] = _stats(x1)


def _nk_mid_body(p_ref, x_ref, st_ref, root, bias, xn_ref, stn_ref):
    agg = (p_ref[0] + p_ref[1])[:_N]
    x = x_ref[...]
    xn = (x - st_ref[0:1, :]) * st_ref[1:2, :]
    x_new = jnp.maximum(
        agg + jnp.dot(xn, root[...], preferred_element_type=_f32)
        + bias[...], 0.0) + x
    xn_ref[...] = x_new
    stn_ref[...] = _stats(x_new)


def _nk_fin_body(p_ref, x_ref, st_ref, root, bias, xf_ref):
    agg = (p_ref[0] + p_ref[1])[:_N, 0:1]
    xn = (x_ref[...] - st_ref[0:1, :]) * st_ref[1:2, :]
    xf = jnp.maximum(
        agg + jnp.dot(xn, root[...], preferred_element_type=_f32)
        + bias[...], 0.0)
    xf_ref[...] = jnp.concatenate(
        [xf, jnp.zeros((_N, _DIM - 1), _f32)], axis=1)


def _whole_call(body, ins, out_shapes):
    return pl.pallas_call(
        body, out_shape=[jax.ShapeDtypeStruct(s, _f32) for s in out_shapes],
    )(*ins)


def _nn_params(p):
    n = p['nn']
    return (n['w1'], n['b1'].reshape(1, -1), n['w2'], n['b2'].reshape(1, -1),
            n['w3'], n['b3'].reshape(1, -1))


def _se_params(p, xdim):
    w1 = p['w1']
    return (w1[:xdim], w1[xdim:2 * xdim], w1[2 * xdim:],
            p['b1'].reshape(1, -1), p['g'].reshape(1, -1),
            p['be'].reshape(1, -1), p['w2'], p['b2'].reshape(1, -1))


def kernel(x, edge_index, edge_attr, params):
    x0 = x.reshape(-1, 1).astype(_f32)
    row = edge_index[0]
    col = edge_index[1]
    pad = _EP - _E
    rowi = jnp.concatenate([row, jnp.zeros((pad,), jnp.int32)]).reshape(-1, _CH)
    coli_g = jnp.concatenate([col, jnp.zeros((pad,), jnp.int32)]).reshape(-1, _CH)
    coli_s = jnp.concatenate(
        [col, jnp.full((pad,), _N, jnp.int32)]).reshape(-1, _CH)
    ea = jnp.concatenate(
        [edge_attr.reshape(-1, 1).astype(_f32), jnp.zeros((pad, 1), _f32)])

    st0 = _whole_call(_s0_body, [x0], [(2, _DIM)])[0]
    x0p = jnp.pad(x0, ((0, 0), (0, _DIM - 1)))
    x0r, _ = _sc_gather_pair(x0p, rowi, coli_g)
    cw = _nn_params(params['conv_in'])
    msg0 = _ek_call(_ek0_body, [ea, x0r], [st0, *cw], [_DIM])[0]
    part = _sc_scatter_add(msg0, coli_s)
    xcur, stcur = _whole_call(
        _nk0_body,
        [part, x0, st0, params['conv_in']['root'],
         params['conv_in']['bias'].reshape(1, -1)],
        [(_N, _DIM), (2, _DIM)])

    se_ps = [_se_params(params['edge_in'], _DIM)] + \
            [_se_params(params['edges'][i], _DIM) for i in range(3)]
    cv_ps = [_nn_params(params['convs'][i]) for i in range(4)]
    rt_ps = [(params['convs'][i]['root'],
              params['convs'][i]['bias'].reshape(1, -1)) for i in range(4)]
    body16 = functools.partial(_ek_mid_body, out_ch=_DIM)
    for i in range(4):
        xr, xc = _sc_gather_pair(xcur, rowi, coli_g)
        ea, msg = _ek_call(body16, [ea, xr, xc],
                           [stcur, *se_ps[i], *cv_ps[i]], [2, _DIM])
        part = _sc_scatter_add(msg, coli_s)
        xcur, stcur = _whole_call(
            _nk_mid_body, [part, xcur, stcur, *rt_ps[i]], [(_N, _DIM), (2, _DIM)])

    xr, xc = _sc_gather_pair(xcur, rowi, coli_g)
    body1 = functools.partial(_ek_mid_body, out_ch=1)
    ea, msg = _ek_call(body1, [ea, xr, xc],
                       [stcur, *_se_params(params['edges'][3], _DIM),
                        *_nn_params(params['conv_out'])], [2, _DIM])
    part = _sc_scatter_add(msg, coli_s)
    xfp = _whole_call(
        _nk_fin_body,
        [part, xcur, stcur, params['conv_out']['root'],
         params['conv_out']['bias'].reshape(1, -1)], [(_N, _DIM)])[0]

    xr, xc = _sc_gather_pair(xfp, rowi, coli_g)
    eaf = _ek_call(_ek_fin_body, [ea, xr, xc],
                   [*_se_params(params['edge_out'], 1)], [1])[0]

    return (xfp[:, 0:1], eaf[:_E])


# speedup vs baseline: 7.9603x; 3.8127x over previous
import functools

import jax
import jax.numpy as jnp
import numpy as np
from jax import lax
from jax.experimental import pallas as pl
from jax.experimental.pallas import tpu as pltpu
from jax.experimental.pallas import tpu_sc as plsc

_DIM = 16
_N = 10000
_E = 160000
_NW = 32
_CH = 128
_CPW = 40
_EPW = _CH * _CPW
_EP = _NW * _EPW
_NACC = 10016
_RPS = _NACC // 16
_EV = _EP // 8
_NV = _N // 8
_PV = _NACC // 8
_V = 512
_G = _EV // _V

_f32 = jnp.float32

_I8 = np.eye(8, dtype=np.float32)


def _k8(w):
    return jnp.kron(jnp.eye(8, dtype=_f32), w)


def _t8(b):
    return jnp.tile(b.reshape(1, -1), (1, 8))


_MEAN16 = np.kron(_I8, np.full((16, 16), 1.0 / 16, np.float32))
_REPL = np.kron(_I8, np.kron(np.eye(16, dtype=np.float32),
                             np.ones((1, 16), np.float32)))
_SELL = np.kron(_I8, np.kron(np.ones((16, 1), np.float32),
                             np.eye(16, dtype=np.float32)))
_S16 = np.kron(_I8, np.ones((16, 1), np.float32))
_E0 = np.zeros((1, 16), np.float32)
_E0[0, 0] = 1.0
_PAD0 = np.kron(_I8, _E0)
_BCAST16 = np.kron(_I8, np.ones((1, 16), np.float32))
_M0 = np.zeros((16, 16), np.float32)
_M0[0, :] = 1.0
_P0R = np.kron(_I8, _M0)
_PICKC0 = np.zeros((16, 1), np.float32)
_PICKC0[0, 0] = 1.0
_AGGP = np.kron(_I8, _PICKC0)
_FOLD = np.kron(np.ones((8, 1), np.float32) / 8,
                np.eye(16, dtype=np.float32))
_SPREAD = np.kron(np.ones((1, 8), np.float32),
                  np.eye(16, dtype=np.float32))


def _mesh():
    return plsc.VectorSubcoreMesh(core_axis_name="c", subcore_axis_name="s",
                                  num_cores=2, num_subcores=16)


def _sc_gather_pair(table, rowi, coli):
    @functools.partial(
        pl.kernel,
        out_type=(jax.ShapeDtypeStruct((_EP, _DIM), _f32),
                  jax.ShapeDtypeStruct((_EP, _DIM), _f32)),
        mesh=_mesh(),
        scratch_types=(pltpu.VMEM((_CPW, _CH), jnp.int32),
                       pltpu.VMEM((_EPW, _DIM), _f32),
                       pltpu.SemaphoreType.DMA),
        compiler_params=pltpu.CompilerParams(use_tc_tiling_on_sc=False),
    )
    def gk(table_h, rowi_h, coli_h, orow_h, ocol_h, idx_v, buf_v, sem):
        cid = lax.axis_index("c")
        sid = lax.axis_index("s")
        w = sid * 2 + cid
        rbase = w * _CPW
        ebase = w * _EPW

        def one(src_h, dst_h):
            pltpu.sync_copy(src_h.at[pl.ds(rbase, _CPW)], idx_v)

            def chunk8(i, carry):
                b = i * 8
                hs = []
                for k in range(8):
                    hs.append(pltpu.async_copy(
                        table_h.at[idx_v.at[b + k]],
                        buf_v.at[pl.ds((b + k) * _CH, _CH)], sem))
                for h in hs:
                    h.wait()
                return carry

            lax.fori_loop(0, _CPW // 8, chunk8, 0)
            pltpu.sync_copy(buf_v, dst_h.at[pl.ds(ebase, _EPW)])

        one(rowi_h, orow_h)
        one(coli_h, ocol_h)

    return gk(table, rowi, coli)


def _sc_scatter_add(msg, coli):
    @functools.partial(
        pl.kernel,
        out_type=jax.ShapeDtypeStruct((2, _NACC, _DIM), _f32),
        mesh=_mesh(),
        scratch_types=(pltpu.VMEM((_CPW, _CH), jnp.int32),
                       pltpu.VMEM((_EPW, _DIM), _f32),
                       pltpu.VMEM((_RPS, _DIM), _f32),
                       pltpu.VMEM_SHARED((_NACC, _DIM), _f32),
                       pltpu.SemaphoreType.DMA),
        compiler_params=pltpu.CompilerParams(use_tc_tiling_on_sc=False),
    )
    def sk(msg_h, coli_h, out_h, idx_v, buf_v, zbuf_v, acc_sh, sem):
        cid = lax.axis_index("c")
        sid = lax.axis_index("s")
        w = sid * 2 + cid

        def zrow(i, carry):
            zbuf_v[i, :] = jnp.zeros((_DIM,), _f32)
            return carry

        lax.fori_loop(0, _RPS, zrow, 0)
        pltpu.sync_copy(zbuf_v, acc_sh.at[pl.ds(sid * _RPS, _RPS)])
        plsc.subcore_barrier()

        pltpu.sync_copy(coli_h.at[pl.ds(w * _CPW, _CPW)], idx_v)
        pltpu.sync_copy(msg_h.at[pl.ds(w * _EPW, _EPW)], buf_v)

        def chunk(i, carry):
            pltpu.sync_copy(buf_v.at[pl.ds(i * _CH, _CH)],
                            acc_sh.at[idx_v.at[i]], add=True)
            return carry

        lax.fori_loop(0, _CPW, chunk, 0)
        plsc.subcore_barrier()

        @pl.when(sid == 0)
        def _():
            pltpu.sync_copy(acc_sh, out_h.at[cid])

    return sk(msg, coli)


def _dot(a, b):
    return jnp.dot(a, b, preferred_element_type=_f32)


def _full_spec(a):
    return pl.BlockSpec(a.shape, lambda i: (0,) * a.ndim)


def _tile_spec(d):
    return pl.BlockSpec((_V, d), lambda i: (i, 0))


def _ek_call(body, tiled_ins, full_ins, out_dims):
    in_specs = ([_tile_spec(a.shape[1]) for a in tiled_ins]
                + [_full_spec(a) for a in full_ins])
    out_shape = [jax.ShapeDtypeStruct((_EV, d), _f32) for d in out_dims]
    out_specs = [_tile_spec(d) for d in out_dims]
    return pl.pallas_call(
        body, grid=(_G,), in_specs=in_specs,
        out_specs=out_specs, out_shape=out_shape,
    )(*tiled_ins, *full_ins)


def _whole_call(body, ins, out_shapes):
    return pl.pallas_call(
        body, out_shape=[jax.ShapeDtypeStruct(s, _f32) for s in out_shapes],
    )(*ins)


def _ek0_body(ea_ref, xr_ref, st_ref, cw1, cb1, cw2, cb2, cw3, cb3, p0r,
              msg_ref):
    e = jnp.abs(ea_ref[...])
    h = jnp.maximum(_dot(e, cw1[...]) + cb1[...], 0.0)
    h = jnp.maximum(_dot(h, cw2[...]) + cb2[...], 0.0)
    w3o = jnp.maximum(_dot(h, cw3[...]) + cb3[...], 0.0)
    xs = _dot((xr_ref[...] - st_ref[0:1, :]) * st_ref[1:2, :], p0r[...])
    msg_ref[...] = w3o * xs


def _ln_block(h, mean16, g8, be8):
    m = _dot(h, mean16)
    hc = h - m
    v = _dot(hc * hc, mean16)
    return hc * lax.rsqrt(v + 1e-5) * g8 + be8


def _ek_mid_body(ea_ref, xr_ref, xc_ref, st_ref,
                 sw1r, sw1c, sw1e, sb1, g8, be8, sw2, sb2, rext, mean16,
                 cw1, cb1, cw2, cb2, cw3, cb3, ctr_a, ctr_b,
                 ea_out_ref, msg_ref, *, out_ch):
    xr = xr_ref[...]
    ea_prev = ea_ref[...]
    h = jnp.maximum(_dot(xr, sw1r[...]) + _dot(xc_ref[...], sw1c[...])
                    + _dot(ea_prev, sw1e[...]) + sb1[...], 0.0)
    h = _ln_block(h, mean16[...], g8[...], be8[...])
    ea_new = jnp.maximum(_dot(h, sw2[...]) + sb2[...], 0.0) \
        + _dot(ea_prev, rext[...])
    ea_out_ref[...] = ea_new
    h1 = jnp.maximum(_dot(ea_new, cw1[...]) + cb1[...], 0.0)
    h2 = jnp.maximum(_dot(h1, cw2[...]) + cb2[...], 0.0)
    w3o = jnp.maximum(_dot(h2, cw3[...]) + cb3[...], 0.0)
    xn = (xr - st_ref[0:1, :]) * st_ref[1:2, :]
    if out_ch == _DIM:
        msg_ref[...] = _dot(_dot(xn, ctr_a[...]) * w3o, ctr_b[...])
    else:
        msg_ref[...] = _dot(_dot(xn * w3o, ctr_a[...]), ctr_b[...])


def _ek_fin_body(ea_ref, xr_ref, xc_ref,
                 sw1r, sw1c, sw1e, sb1, g8, be8, sw2, sb2, mean16,
                 ea_out_ref):
    h = jnp.maximum(_dot(xr_ref[...], sw1r[...])
                    + _dot(xc_ref[...], sw1c[...])
                    + _dot(ea_ref[...], sw1e[...]) + sb1[...], 0.0)
    h = _ln_block(h, mean16[...], g8[...], be8[...])
    ea_out_ref[...] = jnp.maximum(_dot(h, sw2[...]) + sb2[...], 0.0)


def _stats128(x, fold, spread):
    m = _dot(jnp.mean(x, axis=0, keepdims=True), fold)
    m128 = _dot(m, spread)
    d = x - m128
    v = _dot(_dot(jnp.mean(d * d, axis=0, keepdims=True), fold), spread)
    return jnp.concatenate([m128, lax.rsqrt(v + 1e-5)], axis=0)


def _s0_body(x_ref, st_ref):
    x = x_ref[...]
    m = jnp.mean(x)
    v = jnp.mean((x - m) ** 2)
    inv = lax.rsqrt(v + 1e-5)
    st_ref[...] = jnp.concatenate(
        [jnp.full((1, 128), m, _f32), jnp.full((1, 128), inv, _f32)], axis=0)


def _nk0_body(p_ref, x_ref, st_ref, root8, bias8, bc16, fold, spread,
              xn_ref, stn_ref):
    agg = (p_ref[0] + p_ref[1])[:_NV]
    x0 = x_ref[...]
    xn0 = (x0 - st_ref[0, 0]) * st_ref[1, 0]
    x1 = jnp.maximum(agg + _dot(xn0, root8[...]) + bias8[...], 0.0) \
        + _dot(x0, bc16[...])
    xn_ref[...] = x1
    stn_ref[...] = _stats128(x1, fold[...], spread[...])


def _nk_mid_body(p_ref, x_ref, st_ref, root, bias8, fold, spread,
                 xn_ref, stn_ref):
    agg = (p_ref[0] + p_ref[1])[:_NV]
    x = x_ref[...]
    xn = (x - st_ref[0:1, :]) * st_ref[1:2, :]
    x_new = jnp.maximum(agg + _dot(xn, root[...]) + bias8[...], 0.0) + x
    xn_ref[...] = x_new
    stn_ref[...] = _stats128(x_new, fold[...], spread[...])


def _nk_fin_body(p_ref, x_ref, st_ref, root8, bias8, aggp, pad0,
                 xf8_ref, xfp_ref):
    agg = _dot((p_ref[0] + p_ref[1])[:_NV], aggp[...])
    xn = (x_ref[...] - st_ref[0:1, :]) * st_ref[1:2, :]
    xf8 = jnp.maximum(agg + _dot(xn, root8[...]) + bias8[...], 0.0)
    xf8_ref[...] = xf8
    xfp_ref[...] = _dot(xf8, pad0[...])


def _nn_lift(p):
    n = p['nn']
    return (_k8(n['w1']), _t8(n['b1']), _k8(n['w2']), _t8(n['b2']),
            _k8(n['w3']), _t8(n['b3']))


def _se_lift(p, xdim):
    w1 = p['w1']
    return (_k8(w1[:xdim]), _k8(w1[xdim:2 * xdim]), _k8(w1[2 * xdim:]),
            _t8(p['b1']), _t8(p['g']), _t8(p['be']),
            _k8(p['w2']), _t8(p['b2']))


def kernel(x, edge_index, edge_attr, params):
    x0 = x.reshape(-1, 1).astype(_f32)
    row = edge_index[0]
    col = edge_index[1]
    pad = _EP - _E
    rowi = jnp.concatenate([row, jnp.zeros((pad,), jnp.int32)]).reshape(-1, _CH)
    coli_g = jnp.concatenate([col, jnp.zeros((pad,), jnp.int32)]).reshape(-1, _CH)
    coli_s = jnp.concatenate(
        [col, jnp.full((pad,), _N, jnp.int32)]).reshape(-1, _CH)
    ea = jnp.concatenate(
        [edge_attr.reshape(-1, 1).astype(_f32),
         jnp.zeros((pad, 1), _f32)]).reshape(_EV, 8)
    x0v = x0.reshape(_NV, 8)

    st = _whole_call(_s0_body, [x0v], [(2, 128)])[0]
    x0p = jnp.pad(x0, ((0, 0), (0, _DIM - 1)))
    x0r, _ = _sc_gather_pair(x0p, rowi, coli_g)
    cin = params['conv_in']
    msg = _ek_call(_ek0_body, [ea, x0r.reshape(_EV, 128)],
                   [st, *_nn_lift(cin), _P0R], [128])[0]
    part = _sc_scatter_add(msg.reshape(_EP, _DIM), coli_s)
    xcur, st = _whole_call(
        _nk0_body,
        [part.reshape(2, _PV, 128), x0v, st, _k8(cin['root']),
         _t8(cin['bias']), _BCAST16, _FOLD, _SPREAD],
        [(_NV, 128), (2, 128)])

    se_ps = [_se_lift(params['edge_in'], _DIM)] + \
            [_se_lift(params['edges'][i], _DIM) for i in range(3)]
    body16 = functools.partial(_ek_mid_body, out_ch=_DIM)
    for i in range(4):
        cv = params['convs'][i]
        xr, xc = _sc_gather_pair(xcur.reshape(_N, _DIM), rowi, coli_g)
        rext = _k8(jnp.ones((1, 2), _f32)) if i == 0 else _k8(jnp.eye(2, dtype=_f32))
        ea, msg = _ek_call(
            body16,
            [ea, xr.reshape(_EV, 128), xc.reshape(_EV, 128)],
            [st, *se_ps[i], rext, _MEAN16, *_nn_lift(cv), _REPL, _SELL],
            [16, 128])
        part = _sc_scatter_add(msg.reshape(_EP, _DIM), coli_s)
        xcur, st = _whole_call(
            _nk_mid_body,
            [part.reshape(2, _PV, 128), xcur, st, _k8(cv['root']),
             _t8(cv['bias']), _FOLD, _SPREAD],
            [(_NV, 128), (2, 128)])

    cout = params['conv_out']
    xr, xc = _sc_gather_pair(xcur.reshape(_N, _DIM), rowi, coli_g)
    body1 = functools.partial(_ek_mid_body, out_ch=1)
    ea, msg = _ek_call(
        body1,
        [ea, xr.reshape(_EV, 128), xc.reshape(_EV, 128)],
        [st, *_se_lift(params['edges'][3], _DIM), _k8(jnp.eye(2, dtype=_f32)),
         _MEAN16, *_nn_lift(cout), _S16, _PAD0],
        [16, 128])
    part = _sc_scatter_add(msg.reshape(_EP, _DIM), coli_s)
    xf8, xfp = _whole_call(
        _nk_fin_body,
        [part.reshape(2, _PV, 128), xcur, st, _k8(cout['root']),
         _t8(cout['bias']), _AGGP, _PAD0],
        [(_NV, 8), (_NV, 128)])

    eo = params['edge_out']
    w1 = eo['w1']
    sw1r = _k8(jnp.outer(_PICKC0[:, 0], w1[0]))
    sw1c = _k8(jnp.outer(_PICKC0[:, 0], w1[1]))
    sw1e = _k8(w1[2:4])
    xr, xc = _sc_gather_pair(xfp.reshape(_N, _DIM), rowi, coli_g)
    eaf = _ek_call(
        _ek_fin_body,
        [ea, xr.reshape(_EV, 128), xc.reshape(_EV, 128)],
        [sw1r, sw1c, sw1e, _t8(eo['b1']), _t8(eo['g']), _t8(eo['be']),
         _k8(eo['w2']), _t8(eo['b2']), _MEAN16],
        [8])[0]

    return (xf8.reshape(_N, 1), eaf.reshape(_EP, 1)[:_E])
